# baseline (speedup 1.0000x reference)
import jax
import jax.numpy as jnp
from jax import lax
from jax.experimental import pallas as pl
from jax.experimental.pallas import tpu as pltpu

_ROWS = 4096
_COLS = 512
_BLOCK_ROWS = 4096
_N_BLOCKS = _ROWS // _BLOCK_ROWS
_N_TOTAL = float(_ROWS * _COLS)


def _body(x_ref, y_ref, loss_ref, a0, a1, a2):
    i = pl.program_id(0)

    @pl.when(i == 0)
    def _init():
        a0[...] = jnp.zeros((8, _COLS), jnp.float32)
        a1[...] = jnp.zeros((8, _COLS), jnp.float32)
        a2[...] = jnp.zeros((8, _COLS), jnp.float32)

    x = x_ref[...]
    y = y_ref[...]
    per = jnp.maximum(x, 0.0) - x * y + jnp.log(1.0 + jnp.exp(-jnp.abs(x)))
    ones = jnp.ones((8, _BLOCK_ROWS), jnp.float32)
    dn = (((1,), (0,)), ((), ()))
    a0[...] += lax.dot_general(ones, per, dn,
                               preferred_element_type=jnp.float32)
    a1[...] += lax.dot_general(ones, per * y, dn,
                               preferred_element_type=jnp.float32)
    a2[...] += lax.dot_general(ones, y, dn,
                               preferred_element_type=jnp.float32)

    @pl.when(i == _N_BLOCKS - 1)
    def _fin():
        tot = jnp.sum(a0[0:1, :])
        fg = jnp.sum(a1[0:1, :])
        cnt = jnp.sum(a2[0:1, :])
        bg_cnt = jnp.maximum(_N_TOTAL - cnt, 1.0)
        fg_cnt = jnp.maximum(cnt, 1.0)
        loss_ref[0, 0] = (tot - fg) / bg_cnt + fg / fg_cnt


def kernel(output, target):
    x = output.reshape(_ROWS, _COLS)
    y = target.reshape(_ROWS, _COLS)
    loss = pl.pallas_call(
        _body,
        grid=(_N_BLOCKS,),
        in_specs=[
            pl.BlockSpec((_BLOCK_ROWS, _COLS), lambda i: (i, 0)),
            pl.BlockSpec((_BLOCK_ROWS, _COLS), lambda i: (i, 0)),
        ],
        out_specs=pl.BlockSpec(memory_space=pltpu.SMEM),
        out_shape=jax.ShapeDtypeStruct((1, 1), jnp.float32),
        scratch_shapes=[
            pltpu.VMEM((8, _COLS), jnp.float32),
            pltpu.VMEM((8, _COLS), jnp.float32),
            pltpu.VMEM((8, _COLS), jnp.float32),
        ],
    )(x, y)
    return loss[0, 0]


# speedup vs baseline: 1.1032x; 1.1032x over previous
import jax
import jax.numpy as jnp
from jax import lax
from jax.experimental import pallas as pl
from jax.experimental.pallas import tpu as pltpu

_ROWS = 4096
_COLS = 512
_CH_ROWS = 512
_NCH = _ROWS // _CH_ROWS
_N_TOTAL = float(_ROWS * _COLS)


def _body(x_hbm, y_hbm, loss_ref, xb, yb, a0, a1, a2, sems):
    def copies(ci):
        b = ci & 1
        r0 = ci * _CH_ROWS
        return (
            pltpu.make_async_copy(
                x_hbm.at[pl.ds(r0, _CH_ROWS)], xb.at[b], sems.at[0, b]),
            pltpu.make_async_copy(
                y_hbm.at[pl.ds(r0, _CH_ROWS)], yb.at[b], sems.at[1, b]),
        )

    for c in copies(0):
        c.start()

    ones = jnp.ones((8, _CH_ROWS), jnp.float32)
    dn = (((1,), (0,)), ((), ()))
    t = jnp.zeros((8, _COLS), jnp.float32)
    f = jnp.zeros((8, _COLS), jnp.float32)
    cn = jnp.zeros((8, _COLS), jnp.float32)

    for ci in range(_NCH):
        if ci + 1 < _NCH:
            for c in copies(ci + 1):
                c.start()
        for c in copies(ci):
            c.wait()
        b = ci & 1
        x = xb[b]
        y = yb[b]
        per = jnp.maximum(x, 0.0) - x * y + jnp.log(1.0 + jnp.exp(-jnp.abs(x)))
        t = t + lax.dot_general(ones, per, dn,
                                preferred_element_type=jnp.float32)
        f = f + lax.dot_general(ones, per * y, dn,
                                preferred_element_type=jnp.float32)
        cn = cn + lax.dot_general(ones, y, dn,
                                  preferred_element_type=jnp.float32)

    a0[...] = t
    a1[...] = f
    a2[...] = cn
    tot = jnp.sum(a0[0:1, :])
    fg = jnp.sum(a1[0:1, :])
    cnt = jnp.sum(a2[0:1, :])
    bg_cnt = jnp.maximum(_N_TOTAL - cnt, 1.0)
    fg_cnt = jnp.maximum(cnt, 1.0)
    loss_ref[0, 0] = (tot - fg) / bg_cnt + fg / fg_cnt


def kernel(output, target):
    x = output.reshape(_ROWS, _COLS)
    y = target.reshape(_ROWS, _COLS)
    loss = pl.pallas_call(
        _body,
        in_specs=[
            pl.BlockSpec(memory_space=pl.ANY),
            pl.BlockSpec(memory_space=pl.ANY),
        ],
        out_specs=pl.BlockSpec(memory_space=pltpu.SMEM),
        out_shape=jax.ShapeDtypeStruct((1, 1), jnp.float32),
        scratch_shapes=[
            pltpu.VMEM((2, _CH_ROWS, _COLS), jnp.float32),
            pltpu.VMEM((2, _CH_ROWS, _COLS), jnp.float32),
            pltpu.VMEM((8, _COLS), jnp.float32),
            pltpu.VMEM((8, _COLS), jnp.float32),
            pltpu.VMEM((8, _COLS), jnp.float32),
            pltpu.SemaphoreType.DMA((2, 2)),
        ],
    )(x, y)
    return loss[0, 0]


# speedup vs baseline: 1.4386x; 1.3040x over previous
import jax
import jax.numpy as jnp
from jax import lax
from jax.experimental import pallas as pl
from jax.experimental.pallas import tpu as pltpu

_ROWS = 4096
_COLS = 512
_CH_ROWS = 512
_NCH = _ROWS // _CH_ROWS
_N_TOTAL = float(_ROWS * _COLS)


def _body(x_hbm, y_hbm, loss_ref, xb, yb, a0, a1, a2, sems):
    def copies(ci):
        b = ci % 4
        r0 = ci * _CH_ROWS
        return (
            pltpu.make_async_copy(
                x_hbm.at[pl.ds(r0, _CH_ROWS)], xb.at[b], sems.at[0, b]),
            pltpu.make_async_copy(
                y_hbm.at[pl.ds(r0, _CH_ROWS)], yb.at[b], sems.at[1, b]),
        )

    for pf in range(3):
        for c in copies(pf):
            c.start()

    ones = jnp.ones((8, _CH_ROWS), jnp.float32)
    dn = (((1,), (0,)), ((), ()))
    t = jnp.zeros((8, _COLS), jnp.float32)
    f = jnp.zeros((8, _COLS), jnp.float32)
    cn = jnp.zeros((8, _COLS), jnp.float32)

    for ci in range(_NCH):
        if ci + 3 < _NCH:
            for c in copies(ci + 3):
                c.start()
        for c in copies(ci):
            c.wait()
        b = ci % 4
        x = xb[b]
        y = yb[b]
        per = jnp.maximum(x, 0.0) - x * y + jnp.log(1.0 + jnp.exp(-jnp.abs(x)))
        t = t + lax.dot_general(ones, per, dn,
                                preferred_element_type=jnp.float32)
        f = f + lax.dot_general(ones, per * y, dn,
                                preferred_element_type=jnp.float32)
        cn = cn + lax.dot_general(ones, y, dn,
                                  preferred_element_type=jnp.float32)

    a0[...] = t
    a1[...] = f
    a2[...] = cn
    tot = jnp.sum(a0[0:1, :])
    fg = jnp.sum(a1[0:1, :])
    cnt = jnp.sum(a2[0:1, :])
    bg_cnt = jnp.maximum(_N_TOTAL - cnt, 1.0)
    fg_cnt = jnp.maximum(cnt, 1.0)
    loss_ref[0, 0] = (tot - fg) / bg_cnt + fg / fg_cnt


def kernel(output, target):
    x = output.reshape(_ROWS, _COLS)
    y = target.reshape(_ROWS, _COLS)
    loss = pl.pallas_call(
        _body,
        in_specs=[
            pl.BlockSpec(memory_space=pl.ANY),
            pl.BlockSpec(memory_space=pl.ANY),
        ],
        out_specs=pl.BlockSpec(memory_space=pltpu.SMEM),
        out_shape=jax.ShapeDtypeStruct((1, 1), jnp.float32),
        scratch_shapes=[
            pltpu.VMEM((4, _CH_ROWS, _COLS), jnp.float32),
            pltpu.VMEM((4, _CH_ROWS, _COLS), jnp.float32),
            pltpu.VMEM((8, _COLS), jnp.float32),
            pltpu.VMEM((8, _COLS), jnp.float32),
            pltpu.VMEM((8, _COLS), jnp.float32),
            pltpu.SemaphoreType.DMA((2, 4)),
        ],
    )(x, y)
    return loss[0, 0]


# speedup vs baseline: 1.4801x; 1.0289x over previous
import jax
import jax.numpy as jnp
from jax import lax
from jax.experimental import pallas as pl
from jax.experimental.pallas import tpu as pltpu

_ROWS = 4096
_COLS = 512
_CH_ROWS = 256
_NCH = _ROWS // _CH_ROWS
_RING = 8
_PF = 6
_N_TOTAL = float(_ROWS * _COLS)


def _body(x_hbm, y_hbm, loss_ref, xb, yb, a0, a1, a2, sems):
    def copies(ci):
        b = ci % _RING
        r0 = ci * _CH_ROWS
        return (
            pltpu.make_async_copy(
                x_hbm.at[pl.ds(r0, _CH_ROWS)], xb.at[b], sems.at[0, b]),
            pltpu.make_async_copy(
                y_hbm.at[pl.ds(r0, _CH_ROWS)], yb.at[b], sems.at[1, b]),
        )

    for pf in range(_PF):
        for c in copies(pf):
            c.start()

    ones = jnp.ones((8, _CH_ROWS), jnp.float32)
    dn = (((1,), (0,)), ((), ()))
    t = jnp.zeros((8, _COLS), jnp.float32)
    f = jnp.zeros((8, _COLS), jnp.float32)
    cn = jnp.zeros((8, _COLS), jnp.float32)

    for ci in range(_NCH):
        if ci + _PF < _NCH:
            for c in copies(ci + _PF):
                c.start()
        for c in copies(ci):
            c.wait()
        b = ci % _RING
        x = xb[b]
        y = yb[b]
        per = jnp.maximum(x, 0.0) - x * y + jnp.log(1.0 + jnp.exp(-jnp.abs(x)))
        t = t + lax.dot_general(ones, per, dn,
                                preferred_element_type=jnp.float32)
        f = f + lax.dot_general(ones, per * y, dn,
                                preferred_element_type=jnp.float32)
        cn = cn + lax.dot_general(ones, y, dn,
                                  preferred_element_type=jnp.float32)

    a0[...] = t
    a1[...] = f
    a2[...] = cn
    tot = jnp.sum(a0[0:1, :])
    fg = jnp.sum(a1[0:1, :])
    cnt = jnp.sum(a2[0:1, :])
    bg_cnt = jnp.maximum(_N_TOTAL - cnt, 1.0)
    fg_cnt = jnp.maximum(cnt, 1.0)
    loss_ref[0, 0] = (tot - fg) / bg_cnt + fg / fg_cnt


def kernel(output, target):
    x = output.reshape(_ROWS, _COLS)
    y = target.reshape(_ROWS, _COLS)
    loss = pl.pallas_call(
        _body,
        in_specs=[
            pl.BlockSpec(memory_space=pl.ANY),
            pl.BlockSpec(memory_space=pl.ANY),
        ],
        out_specs=pl.BlockSpec(memory_space=pltpu.SMEM),
        out_shape=jax.ShapeDtypeStruct((1, 1), jnp.float32),
        scratch_shapes=[
            pltpu.VMEM((_RING, _CH_ROWS, _COLS), jnp.float32),
            pltpu.VMEM((_RING, _CH_ROWS, _COLS), jnp.float32),
            pltpu.VMEM((8, _COLS), jnp.float32),
            pltpu.VMEM((8, _COLS), jnp.float32),
            pltpu.VMEM((8, _COLS), jnp.float32),
            pltpu.SemaphoreType.DMA((2, _RING)),
        ],
    )(x, y)
    return loss[0, 0]
